# baseline (speedup 1.0000x reference)
import functools

import jax
import jax.numpy as jnp
from jax import lax
from jax.experimental import pallas as pl
from jax.experimental.pallas import tpu as pltpu
from jax.experimental.pallas import tpu_sc as plsc

BATCH = 16384
NODE = 1000000
HIDDEN = 64
TW = 128
NC = 2
NS = 16
L = 16
NW = NC * NS
BPW = BATCH // NW
CH = 128
NCH = BPW // CH
GPC = CH // L

NCHK = NODE // TW
TAIL = NODE - NCHK * TW
NPT = NCHK // NW + 1
NSUP = (NPT + 1) // 2


def _neg_sqrt(x):
    i = lax.bitcast_convert_type(x, jnp.int32)
    y = lax.bitcast_convert_type(jnp.int32(0x5F3759DF) - (i >> 1), jnp.float32)
    for _ in range(3):
        y = y * (1.5 - 0.5 * x * y * y)
    return -(x * y)


def _mrg(lane, s, a, b):
    pa = a.at[lane ^ s].get(mode="promise_in_bounds")
    pb = b.at[lane ^ s].get(mode="promise_in_bounds")
    return jnp.where((lane & s) == 0, a + pa, b + pb)


@functools.cache
def _build_transpose_kernel():
  mesh = plsc.VectorSubcoreMesh(
      core_axis_name="c", subcore_axis_name="s", num_cores=NC, num_subcores=NS
  )
  vm = pltpu.VMEM

  @functools.partial(
      pl.kernel,
      out_type=jax.ShapeDtypeStruct((NODE * HIDDEN,), jnp.float32),
      mesh=mesh,
      compiler_params=pltpu.CompilerParams(
          use_tc_tiling_on_sc=True, needs_layout_passes=False),
      scratch_types=[
          vm((HIDDEN, TW), jnp.float32), vm((HIDDEN, TW), jnp.float32),
          vm((HIDDEN * TW,), jnp.float32), vm((HIDDEN * TW,), jnp.float32),
          vm((TAIL, HIDDEN), jnp.float32),
          pltpu.SemaphoreType.DMA, pltpu.SemaphoreType.DMA,
          pltpu.SemaphoreType.DMA, pltpu.SemaphoreType.DMA,
      ],
  )
  def _t_kernel(nodet_hbm, tail_hbm, out_hbm, in0, in1, ob0, ob1, tb,
                si0, si1, so0, so1):
      wid = lax.axis_index("s") * NC + lax.axis_index("c")
      lane = lax.iota(jnp.int32, L)
      inb = (in0, in1)
      outb = (ob0, ob1)
      sin = (si0, si1)
      sout = (so0, so1)
      avec = []
      for k in range(TW // L):
          m = k * L + lane
          avec.append(((m >> 1) << 7) + ((m & 1) << 6))

      def start_in(c, slot):
          src = nodet_hbm.at[:, pl.ds(pl.multiple_of(c * TW, TW), TW)]
          pltpu.async_copy(src, inb[slot], sin[slot])

      def chunk_slot(st, slot):
          c = wid + (2 * st + slot) * NW

          @pl.when(c < NCHK)
          def _():
              pltpu.make_async_copy(
                  nodet_hbm.at[:, pl.ds(0, TW)], inb[slot], sin[slot]).wait()

              @pl.when(st > 0)
              def _():
                  pltpu.make_async_copy(
                      outb[slot], out_hbm.at[pl.ds(0, HIDDEN * TW)],
                      sout[slot]).wait()

              for f in range(HIDDEN):
                  for k in range(TW // L):
                      v = inb[slot][f, pl.ds(k * L, L)]
                      plsc.store_scatter(outb[slot], [avec[k] + f], v)

              @pl.when(c + 2 * NW < NCHK)
              def _():
                  start_in(c + 2 * NW, slot)

              pltpu.async_copy(
                  outb[slot],
                  out_hbm.at[pl.ds(c * (HIDDEN * TW), HIDDEN * TW)],
                  sout[slot])

      start_in(wid, 0)
      start_in(wid + NW, 1)

      def super_step(st, carry):
          chunk_slot(st, 0)
          chunk_slot(st, 1)
          return carry

      lax.fori_loop(0, NSUP, super_step, 0)
      for slot in (0, 1):
          pltpu.make_async_copy(
              outb[slot], out_hbm.at[pl.ds(0, HIDDEN * TW)],
              sout[slot]).wait()

      @pl.when(wid == NW - 1)
      def _():
          pltpu.sync_copy(tail_hbm, tb)
          for m in range(TAIL):
              dst = (m >> 1) * TW + (m & 1) * HIDDEN
              for k in range(HIDDEN // L):
                  outb[0][pl.ds(dst + k * L, L)] = tb[m, pl.ds(k * L, L)]
          pltpu.sync_copy(
              outb[0].at[pl.ds(0, TAIL * HIDDEN)],
              out_hbm.at[pl.ds(NCHK * TW * HIDDEN, TAIL * HIDDEN)])

  return _t_kernel


@functools.cache
def _build_gather_kernel():
  mesh = plsc.VectorSubcoreMesh(
      core_axis_name="c", subcore_axis_name="s", num_cores=NC, num_subcores=NS
  )
  vm = pltpu.VMEM

  @functools.partial(
      pl.kernel,
      out_type=jax.ShapeDtypeStruct((BATCH,), jnp.float32),
      mesh=mesh,
      compiler_params=pltpu.CompilerParams(use_tc_tiling_on_sc=True),
      scratch_types=[
          vm((CH,), jnp.int32), vm((CH,), jnp.int32),
          vm((CH,), jnp.int32), vm((CH,), jnp.int32),
          vm((CH,), jnp.int32), vm((CH,), jnp.int32),
          vm((CH,), jnp.int32), vm((CH,), jnp.int32),
          vm((CH,), jnp.int32), vm((CH,), jnp.int32),
          vm((CH,), jnp.int32), vm((CH,), jnp.int32),
          vm((CH, TW), jnp.float32), vm((CH, TW), jnp.float32),
          vm((CH, TW), jnp.float32), vm((CH, TW), jnp.float32),
          vm((CH, TW), jnp.float32), vm((CH, TW), jnp.float32),
          vm((CH,), jnp.float32),
          pltpu.SemaphoreType.DMA,
          pltpu.SemaphoreType.DMA,
      ],
  )
  def _sc_kernel(head_hbm, rel_hbm, tail_hbm, nodep_hbm, relp_hbm, out_hbm,
                 hi0, hi1, ri0, ri1, ti0, ti1,
                 hs0, hs1, rs0, rs1, ts0, ts1,
                 hb0, hb1, rb0, rb1, tb0, tb1, osc, s0, s1):
      wid = lax.axis_index("s") * NC + lax.axis_index("c")
      base = pl.multiple_of(wid * BPW, BPW)
      lane = lax.iota(jnp.int32, L)
      hidx, ridx, tidx = (hi0, hi1), (ri0, ri1), (ti0, ti1)
      hsm, rsm, tsm = (hs0, hs1), (rs0, rs1), (ts0, ts1)
      hbuf, rbuf, tbuf = (hb0, hb1), (rb0, rb1), (tb0, tb1)
      sems = (s0, s1)

      def start_chunk(c, slot):
          cb = pl.multiple_of(base + c * CH, CH)
          pltpu.sync_copy(head_hbm.at[pl.ds(cb, CH)], hidx[slot])
          pltpu.sync_copy(rel_hbm.at[pl.ds(cb, CH)], ridx[slot])
          pltpu.sync_copy(tail_hbm.at[pl.ds(cb, CH)], tidx[slot])

          def halve(i, carry):
              off = i * L
              for idxb, parb in ((hidx, hsm), (ridx, rsm), (tidx, tsm)):
                  v = idxb[slot][pl.ds(off, L)]
                  parb[slot][pl.ds(off, L)] = (v & 1) * HIDDEN
                  idxb[slot][pl.ds(off, L)] = v >> 1
              return carry

          lax.fori_loop(0, CH // L, halve, 0)
          sem = sems[slot]
          return (
              pltpu.async_copy(nodep_hbm.at[hidx[slot]], hbuf[slot], sem),
              pltpu.async_copy(relp_hbm.at[ridx[slot]], rbuf[slot], sem),
              pltpu.async_copy(nodep_hbm.at[tidx[slot]], tbuf[slot], sem),
          )

      inflight = {0: start_chunk(0, 0)}
      inflight[1] = start_chunk(1, 1)

      for c in range(NCH):
          slot = c & 1
          for cp in inflight[slot]:
              cp.wait()
          hb, rb, tb = hbuf[slot], rbuf[slot], tbuf[slot]
          hs, rs, ts = hsm[slot], rsm[slot], tsm[slot]

          def group_body(g, carry, hb=hb, rb=rb, tb=tb, hs=hs, rs=rs, ts=ts):
              gbase = g * L
              hso = hs[pl.ds(gbase, L)]
              rso = rs[pl.ds(gbase, L)]
              tso = ts[pl.ds(gbase, L)]

              def rowacc(j):
                  row = gbase + j
                  ho = hso[j]
                  ro = rso[j]
                  to = tso[j]
                  acc = None
                  for cc in range(HIDDEN // L):
                      hv = hb[row, pl.ds(ho + cc * L, L)]
                      rv = rb[row, pl.ds(ro + cc * L, L)]
                      tv = tb[row, pl.ds(to + cc * L, L)]
                      dd = (hv + rv) - tv
                      sq = dd * dd
                      acc = sq if acc is None else acc + sq
                  return acc

              def quad(r):
                  c_lo = _mrg(lane, 8, rowacc(r), rowacc(r + 8))
                  c_hi = _mrg(lane, 8, rowacc(r + 4), rowacc(r + 12))
                  return _mrg(lane, 4, c_lo, c_hi)

              e0 = _mrg(lane, 2, quad(0), quad(2))
              e1 = _mrg(lane, 2, quad(1), quad(3))
              tot = _mrg(lane, 1, e0, e1)
              osc[pl.ds(gbase, L)] = _neg_sqrt(tot + 1e-12)
              return carry

          lax.fori_loop(0, GPC, group_body, 0)
          cb = pl.multiple_of(base + c * CH, CH)
          pltpu.sync_copy(osc, out_hbm.at[pl.ds(cb, CH)])
          if c + 2 < NCH:
              inflight[slot] = start_chunk(c + 2, slot)

  return _sc_kernel


def kernel(head_index, rel_type, tail_index, node_emb, rel_emb):
    flat = _build_transpose_kernel()(node_emb.T, node_emb[NCHK * TW:])
    nodep = flat.reshape(NODE // 2, TW)
    relp = rel_emb.reshape(rel_emb.shape[0] // 2, TW)
    return _build_gather_kernel()(head_index, rel_type, tail_index,
                                  nodep, relp)


# speedup vs baseline: 2.0187x; 2.0187x over previous
import functools

import jax
import jax.numpy as jnp
from jax import lax
from jax.experimental import pallas as pl
from jax.experimental.pallas import tpu as pltpu
from jax.experimental.pallas import tpu_sc as plsc

BATCH = 16384
HIDDEN = 64
NC = 2
NS = 16
L = 16
NW = NC * NS
BPW = BATCH // NW
CH = 128
NCH = BPW // CH
GPC = CH // L


def _neg_sqrt(x):
    i = lax.bitcast_convert_type(x, jnp.int32)
    y = lax.bitcast_convert_type(jnp.int32(0x5F3759DF) - (i >> 1), jnp.float32)
    for _ in range(3):
        y = y * (1.5 - 0.5 * x * y * y)
    return -(x * y)


def _mrg(lane, s, a, b):
    pa = a.at[lane ^ s].get(mode="promise_in_bounds")
    pb = b.at[lane ^ s].get(mode="promise_in_bounds")
    return jnp.where((lane & s) == 0, a + pa, b + pb)


@functools.cache
def _build_sc_kernel():
  mesh = plsc.VectorSubcoreMesh(
      core_axis_name="c", subcore_axis_name="s", num_cores=NC, num_subcores=NS
  )

  @functools.partial(
      pl.kernel,
      out_type=jax.ShapeDtypeStruct((BATCH,), jnp.float32),
      mesh=mesh,
      compiler_params=pltpu.CompilerParams(use_tc_tiling_on_sc=False),
      scratch_types=[
          pltpu.VMEM((CH,), jnp.int32), pltpu.VMEM((CH,), jnp.int32),
          pltpu.VMEM((CH,), jnp.int32), pltpu.VMEM((CH,), jnp.int32),
          pltpu.VMEM((CH,), jnp.int32), pltpu.VMEM((CH,), jnp.int32),
          pltpu.VMEM((CH, HIDDEN), jnp.float32),
          pltpu.VMEM((CH, HIDDEN), jnp.float32),
          pltpu.VMEM((CH, HIDDEN), jnp.float32),
          pltpu.VMEM((CH, HIDDEN), jnp.float32),
          pltpu.VMEM((CH, HIDDEN), jnp.float32),
          pltpu.VMEM((CH, HIDDEN), jnp.float32),
          pltpu.VMEM((CH,), jnp.float32),
          pltpu.VMEM((CH * L // 2,), jnp.float32),
          pltpu.SemaphoreType.DMA,
          pltpu.SemaphoreType.DMA,
      ],
  )
  def _sc_kernel(head_hbm, rel_hbm, tail_hbm, node_hbm, relemb_hbm, out_hbm,
                 hi0, hi1, ri0, ri1, ti0, ti1,
                 hb0, hb1, rb0, rb1, tb0, tb1, osc, pbuf, s0, s1):
      wid = lax.axis_index("s") * NC + lax.axis_index("c")
      base = pl.multiple_of(wid * BPW, BPW)
      lane = lax.iota(jnp.int32, L)
      hidx, ridx, tidx = (hi0, hi1), (ri0, ri1), (ti0, ti1)
      hbuf, rbuf, tbuf = (hb0, hb1), (rb0, rb1), (tb0, tb1)
      sems = (s0, s1)

      def load_idx(c, slot):
          cb = pl.multiple_of(base + c * CH, CH)
          pltpu.sync_copy(head_hbm.at[pl.ds(cb, CH)], hidx[slot])
          pltpu.sync_copy(rel_hbm.at[pl.ds(cb, CH)], ridx[slot])
          pltpu.sync_copy(tail_hbm.at[pl.ds(cb, CH)], tidx[slot])

      def start_gather(slot):
          sem = sems[slot]
          return (
              pltpu.async_copy(node_hbm.at[hidx[slot]], hbuf[slot], sem),
              pltpu.async_copy(relemb_hbm.at[ridx[slot]], rbuf[slot], sem),
              pltpu.async_copy(node_hbm.at[tidx[slot]], tbuf[slot], sem),
          )

      load_idx(0, 0)
      inflight = {0: start_gather(0)}
      load_idx(1, 1)
      inflight[1] = start_gather(1)

      for c in range(NCH):
          slot = c & 1
          for cp in inflight[slot]:
              cp.wait()
          hb, rb, tb = hbuf[slot], rbuf[slot], tbuf[slot]

          def pair_body(p, carry, hb=hb, rb=rb, tb=tb):
              g = p >> 3
              r = p & 7
              rowa = (g << 4) + r

              def rowacc(row):
                  acc = None
                  for cc in range(HIDDEN // L):
                      hv = hb[row, pl.ds(cc * L, L)]
                      rv = rb[row, pl.ds(cc * L, L)]
                      tv = tb[row, pl.ds(cc * L, L)]
                      dd = (hv + rv) - tv
                      sq = dd * dd
                      acc = sq if acc is None else acc + sq
                  return acc

              pbuf[pl.ds(p * L, L)] = _mrg(lane, 8, rowacc(rowa),
                                           rowacc(rowa + 8))
              return carry

          def group_body(g, carry):
              b8 = g * 8
              q = [pbuf[pl.ds((b8 + r) * L, L)] for r in range(8)]
              t = [_mrg(lane, 4, q[r], q[r + 4]) for r in range(4)]
              e0 = _mrg(lane, 2, t[0], t[2])
              e1 = _mrg(lane, 2, t[1], t[3])
              tot = _mrg(lane, 1, e0, e1)
              osc[pl.ds(g * L, L)] = _neg_sqrt(tot + 1e-12)
              return carry

          lax.fori_loop(0, CH // 2, pair_body, 0)
          lax.fori_loop(0, GPC, group_body, 0)
          cb = pl.multiple_of(base + c * CH, CH)
          pltpu.sync_copy(osc, out_hbm.at[pl.ds(cb, CH)])
          if c + 2 < NCH:
              load_idx(c + 2, slot)
              inflight[slot] = start_gather(slot)

  return _sc_kernel


def kernel(head_index, rel_type, tail_index, node_emb, rel_emb):
    return _build_sc_kernel()(head_index, rel_type, tail_index, node_emb, rel_emb)
